# baseline (speedup 1.0000x reference)
import functools
import math

import jax
import jax.numpy as jnp
from jax import lax
from jax.experimental import pallas as pl

B, C, H, W = 16, 100, 32, 32
N = H * W
K = 9
HID = 2 * C
FFN_HID = 4 * C
EPS = 1e-5

_F32 = jnp.float32
_HIGH = lax.Precision.HIGHEST
_BPP = 2


def _dot(a, b, dims, precision=_HIGH):
    return lax.dot_general(a, b, (dims, ((), ())), precision=precision,
                           preferred_element_type=_F32)


def _gelu(u):
    return 0.5 * u * (1.0 + lax.erf(u * (1.0 / math.sqrt(2.0))))


def _vig_body(x_ref, w1_ref, b1_ref, g1_ref, be1_ref, wa_ref, wm_ref, bgc_ref,
              w2_ref, b2_ref, f1_ref, bf1_ref, f2_ref, bf2_ref, out_ref):
    for bi in range(_BPP):
        _vig_one(x_ref[bi], w1_ref, b1_ref, g1_ref, be1_ref, wa_ref, wm_ref,
                 bgc_ref, w2_ref, b2_ref, f1_ref, bf1_ref, f2_ref, bf2_ref,
                 out_ref, bi)


def _vig_one(xb, w1_ref, b1_ref, g1_ref, be1_ref, wa_ref, wm_ref, bgc_ref,
             w2_ref, b2_ref, f1_ref, bf1_ref, f2_ref, bf2_ref, out_ref, bi):
    conv = _dot(xb, w1_ref[...], (((1,), (0,))), precision=lax.Precision.DEFAULT)
    ht = (conv + b1_ref[...]) / jnp.sqrt(_F32(1.0) + _F32(EPS)) * g1_ref[...] \
        + be1_ref[...]

    x2 = jnp.sum(ht * ht, axis=1)
    inner = _dot(ht, ht, (((1,), (1,))), precision=lax.Precision.DEFAULT)
    dist = x2[:, None] - 2.0 * inner + x2[None, :]

    cols = lax.broadcasted_iota(jnp.int32, (N, N), 1)
    relmax = jnp.full((N, C), -jnp.inf, dtype=_F32)
    for _ in range(K):
        m = jnp.min(dist, axis=1, keepdims=True)
        eq = dist == m
        jsel = jnp.min(jnp.where(eq, cols, N), axis=1, keepdims=True)
        oh = cols == jsel
        ohf = oh.astype(_F32)
        sel = _dot(ohf, ht, (((1,), (0,))), precision=lax.Precision.DEFAULT)
        relmax = jnp.maximum(relmax, sel)
        dist = jnp.where(oh, jnp.inf, dist)

    u = (_dot(ht, wa_ref[...], (((1,), (0,))))
         + _dot(relmax, wm_ref[...], (((1,), (0,))))
         + bgc_ref[...])
    u = _gelu(u)
    h4 = _dot(u, w2_ref[...], (((1,), (0,)))) + b2_ref[...]
    y1 = h4 + xb

    v = _gelu(_dot(y1, f1_ref[...], (((1,), (0,)))) + bf1_ref[...])
    y2 = _dot(v, f2_ref[...], (((1,), (0,)))) + bf2_ref[...] + y1
    out_ref[bi] = y2


def kernel(x, g_fc1_w, g_fc1_b, g_bn1_g, g_bn1_b, gc_w, gc_b, gc_bn_g, gc_bn_b,
           g_fc2_w, g_fc2_b, g_bn2_g, g_bn2_b,
           f_fc1_w, f_fc1_b, f_bn1_g, f_bn1_b, f_fc2_w, f_fc2_b, f_bn2_g, f_bn2_b):
    s = 1.0 / jnp.sqrt(jnp.float32(1.0 + EPS))

    def fold(w, b, g, be):
        sc = s * g
        return (w * sc[:, None]).T, (b * sc + be)[None, :]

    w1 = g_fc1_w.T
    b1 = g_fc1_b[None, :]
    g1 = g_bn1_g[None, :]
    be1 = g_bn1_b[None, :]
    gcw, bgc = fold(gc_w, gc_b, gc_bn_g, gc_bn_b)
    wa = gcw[:C] - gcw[C:]
    wm = gcw[C:]
    w2, b2 = fold(g_fc2_w, g_fc2_b, g_bn2_g, g_bn2_b)
    f1, bf1 = fold(f_fc1_w, f_fc1_b, f_bn1_g, f_bn1_b)
    f2, bf2 = fold(f_fc2_w, f_fc2_b, f_bn2_g, f_bn2_b)

    xt = x.reshape(B, C, N).transpose(0, 2, 1)

    def fixed(shape):
        return pl.BlockSpec(shape, lambda b: (0,) * len(shape))

    out = pl.pallas_call(
        _vig_body,
        grid=(B // _BPP,),
        in_specs=[
            pl.BlockSpec((_BPP, N, C), lambda b: (b, 0, 0)),
            fixed((C, C)), fixed((1, C)), fixed((1, C)), fixed((1, C)),
            fixed((C, HID)), fixed((C, HID)), fixed((1, HID)),
            fixed((HID, C)), fixed((1, C)),
            fixed((C, FFN_HID)), fixed((1, FFN_HID)),
            fixed((FFN_HID, C)), fixed((1, C)),
        ],
        out_specs=pl.BlockSpec((_BPP, N, C), lambda b: (b, 0, 0)),
        out_shape=jax.ShapeDtypeStruct((B, N, C), _F32),
    )(xt, w1, b1, g1, be1, wa, wm, bgc, w2, b2, f1, bf1, f2, bf2)

    return out.transpose(0, 2, 1).reshape(B, C, H, W)


# speedup vs baseline: 1.2866x; 1.2866x over previous
import functools
import math

import jax
import jax.numpy as jnp
from jax import lax
from jax.experimental import pallas as pl

B, C, H, W = 16, 100, 32, 32
N = H * W
K = 9
HID = 2 * C
FFN_HID = 4 * C
EPS = 1e-5

_F32 = jnp.float32
_HIGH = lax.Precision.HIGHEST
_BPP = 1


def _dot(a, b, dims, precision=_HIGH):
    return lax.dot_general(a, b, (dims, ((), ())), precision=precision,
                           preferred_element_type=_F32)


def _gelu(u):
    return 0.5 * u * (1.0 + lax.erf(u * (1.0 / math.sqrt(2.0))))


def _vig_body(x_ref, w1_ref, b1_ref, g1_ref, be1_ref, wa_ref, wm_ref, bgc_ref,
              w2_ref, b2_ref, f1_ref, bf1_ref, f2_ref, bf2_ref, out_ref):
    for bi in range(_BPP):
        _vig_one(x_ref[bi], w1_ref, b1_ref, g1_ref, be1_ref, wa_ref, wm_ref,
                 bgc_ref, w2_ref, b2_ref, f1_ref, bf1_ref, f2_ref, bf2_ref,
                 out_ref, bi)


def _vig_one(xb, w1_ref, b1_ref, g1_ref, be1_ref, wa_ref, wm_ref, bgc_ref,
             w2_ref, b2_ref, f1_ref, bf1_ref, f2_ref, bf2_ref, out_ref, bi):
    conv = _dot(xb, w1_ref[...], (((1,), (0,))), precision=lax.Precision.DEFAULT)
    ht = (conv + b1_ref[...]) / jnp.sqrt(_F32(1.0) + _F32(EPS)) * g1_ref[...] \
        + be1_ref[...]

    x2 = jnp.sum(ht * ht, axis=1)
    inner = _dot(ht, ht, (((1,), (1,))), precision=lax.Precision.DEFAULT)
    dist = x2[:, None] - 2.0 * inner + x2[None, :]

    cols = lax.broadcasted_iota(jnp.int32, (N, N), 1)
    relmax = jnp.full((N, C), -jnp.inf, dtype=_F32)
    for _ in range(K):
        m = jnp.min(dist, axis=1, keepdims=True)
        eq = dist == m
        jsel = jnp.min(jnp.where(eq, cols, N), axis=1, keepdims=True)
        oh = cols == jsel
        ohf = oh.astype(_F32)
        sel = _dot(ohf, ht, (((1,), (0,))), precision=lax.Precision.DEFAULT)
        relmax = jnp.maximum(relmax, sel)
        dist = jnp.where(oh, jnp.inf, dist)

    u = (_dot(ht, wa_ref[...], (((1,), (0,))))
         + _dot(relmax, wm_ref[...], (((1,), (0,))))
         + bgc_ref[...])
    u = _gelu(u)
    h4 = _dot(u, w2_ref[...], (((1,), (0,)))) + b2_ref[...]
    y1 = h4 + xb

    v = _gelu(_dot(y1, f1_ref[...], (((1,), (0,)))) + bf1_ref[...])
    y2 = _dot(v, f2_ref[...], (((1,), (0,)))) + bf2_ref[...] + y1
    out_ref[bi] = y2


def kernel(x, g_fc1_w, g_fc1_b, g_bn1_g, g_bn1_b, gc_w, gc_b, gc_bn_g, gc_bn_b,
           g_fc2_w, g_fc2_b, g_bn2_g, g_bn2_b,
           f_fc1_w, f_fc1_b, f_bn1_g, f_bn1_b, f_fc2_w, f_fc2_b, f_bn2_g, f_bn2_b):
    s = 1.0 / jnp.sqrt(jnp.float32(1.0 + EPS))

    def fold(w, b, g, be):
        sc = s * g
        return (w * sc[:, None]).T, (b * sc + be)[None, :]

    w1 = g_fc1_w.T
    b1 = g_fc1_b[None, :]
    g1 = g_bn1_g[None, :]
    be1 = g_bn1_b[None, :]
    gcw, bgc = fold(gc_w, gc_b, gc_bn_g, gc_bn_b)
    wa = gcw[:C] - gcw[C:]
    wm = gcw[C:]
    w2, b2 = fold(g_fc2_w, g_fc2_b, g_bn2_g, g_bn2_b)
    f1, bf1 = fold(f_fc1_w, f_fc1_b, f_bn1_g, f_bn1_b)
    f2, bf2 = fold(f_fc2_w, f_fc2_b, f_bn2_g, f_bn2_b)

    xt = x.reshape(B, C, N).transpose(0, 2, 1)

    def fixed(shape):
        return pl.BlockSpec(shape, lambda b: (0,) * len(shape))

    out = pl.pallas_call(
        _vig_body,
        grid=(B // _BPP,),
        in_specs=[
            pl.BlockSpec((_BPP, N, C), lambda b: (b, 0, 0)),
            fixed((C, C)), fixed((1, C)), fixed((1, C)), fixed((1, C)),
            fixed((C, HID)), fixed((C, HID)), fixed((1, HID)),
            fixed((HID, C)), fixed((1, C)),
            fixed((C, FFN_HID)), fixed((1, FFN_HID)),
            fixed((FFN_HID, C)), fixed((1, C)),
        ],
        out_specs=pl.BlockSpec((_BPP, N, C), lambda b: (b, 0, 0)),
        out_shape=jax.ShapeDtypeStruct((B, N, C), _F32),
    )(xt, w1, b1, g1, be1, wa, wm, bgc, w2, b2, f1, bf1, f2, bf2)

    return out.transpose(0, 2, 1).reshape(B, C, H, W)
